# Initial kernel scaffold; baseline (speedup 1.0000x reference)
import jax
import jax.numpy as jnp
from jax.experimental import pallas as pl


def kernel(x, edge_index, context, W1, b1, g1, bt1, W2, b2, go_g, go_b, W3, b3, g2, bt2, W4, b4, Wr1, br1, Wr2, br2, Wd, bd, Wc, bc):
    raise NotImplementedError("write your pallas kernel here")


# speedup vs baseline: 2.4889x; 2.4889x over previous
import functools

import jax
import jax.numpy as jnp
from jax import lax
from jax.experimental import pallas as pl
from jax.experimental.pallas import tpu as pltpu
from jax.experimental.pallas import tpu_sc as plsc

N = 10000
E = 160000
D_IN = 256
H = 512
C_OUT = 64
CTX_PAD = 8
EXPL = 32

LANES = 16
NUM_TILES = 16
K_EDGE = 128
CH = 79
E_PAD = NUM_TILES * K_EDGE * CH
EPT = K_EDGE * CH
ACC_ROWS = 10240
ZROWS = ACC_ROWS // NUM_TILES
WROWS = 624
WCHUNKS = ((0, 128), (128, 128), (256, 128), (384, 128), (512, 112))
WTAIL = N - NUM_TILES * WROWS

ROW_BLK = 400
GRID = N // ROW_BLK


def _make_segsum(nblk):
    bpc = nblk // 2

    mesh = plsc.VectorSubcoreMesh(core_axis_name="c", subcore_axis_name="s")

    @functools.partial(
        pl.kernel,
        mesh=mesh,
        out_type=jax.ShapeDtypeStruct((nblk * N, 128), jnp.float32),
        scratch_types=[
            pltpu.VMEM((K_EDGE,), jnp.int32),
            pltpu.VMEM((K_EDGE,), jnp.int32),
            pltpu.VMEM((K_EDGE,), jnp.int32),
            pltpu.VMEM((K_EDGE, 128), jnp.float32),
            pltpu.VMEM_SHARED((ACC_ROWS, 128), jnp.float32),
            pltpu.SemaphoreType.DMA,
        ],
    )
    def segsum(x_hbm, src_hbm, dst_hbm, zeros_hbm, out_hbm,
               src_v, sadj_v, dst_v, rows_v, acc, sem):
        c = lax.axis_index("c")
        s = lax.axis_index("s")
        ebase = s * EPT

        for local in range(bpc):
            b = c * bpc + local
            row_off = b * N

            pltpu.sync_copy(zeros_hbm, acc.at[pl.ds(s * ZROWS, ZROWS)])
            plsc.subcore_barrier()

            def edge_chunk(i, carry):
                base = ebase + i * K_EDGE
                pltpu.sync_copy(src_hbm.at[pl.ds(base, K_EDGE)], src_v)
                pltpu.sync_copy(dst_hbm.at[pl.ds(base, K_EDGE)], dst_v)
                for j in range(K_EDGE // LANES):
                    sl = pl.ds(j * LANES, LANES)
                    sadj_v[sl] = src_v[sl] + row_off
                pltpu.async_copy(x_hbm.at[sadj_v], rows_v, sem).wait()
                pltpu.sync_copy(rows_v, acc.at[dst_v], add=True)
                return carry

            lax.fori_loop(0, CH, edge_chunk, 0)
            plsc.subcore_barrier()

            for off, sz in WCHUNKS:
                r0 = s * WROWS + off
                pltpu.sync_copy(acc.at[pl.ds(r0, sz)], rows_v.at[pl.ds(0, sz)])
                pltpu.sync_copy(rows_v.at[pl.ds(0, sz)],
                                out_hbm.at[pl.ds(row_off + r0, sz)])

            @pl.when(s == NUM_TILES - 1)
            def _():
                r0 = NUM_TILES * WROWS
                pltpu.sync_copy(acc.at[pl.ds(r0, WTAIL)],
                                rows_v.at[pl.ds(0, WTAIL)])
                pltpu.sync_copy(rows_v.at[pl.ds(0, WTAIL)],
                                out_hbm.at[pl.ds(row_off + r0, WTAIL)])
            if local + 1 < bpc:
                plsc.subcore_barrier()

    return segsum


@functools.lru_cache(maxsize=None)
def _get_segsum(nblk):
    return _make_segsum(nblk)


def _dot(a, b):
    return jnp.dot(a, b, preferred_element_type=jnp.float32)


def _accum_stats(i, t, acc_ref, stats_ref):
    @pl.when(i == 0)
    def _():
        acc_ref[...] = jnp.zeros_like(acc_ref)

    acc_ref[0:1, :] += jnp.sum(t, axis=0, keepdims=True)
    acc_ref[1:2, :] += jnp.sum(t * t, axis=0, keepdims=True)

    @pl.when(i == pl.num_programs(0) - 1)
    def _():
        stats_ref[...] = acc_ref[...]


def _bn_in(t, stats, g, bt):
    mu = stats[0:1, :] / N
    var = stats[1:2, :] / N - mu * mu
    inv = lax.rsqrt(var + 1e-5)
    return (t - mu) * inv * g + bt


def _k1_body(x_ref, agg_ref, w1_ref, b1_ref, t1_ref, stats_ref, acc_ref):
    i = pl.program_id(0)
    t = _dot(x_ref[:, 0:128] + agg_ref[0], w1_ref[0:128, :])
    t += _dot(x_ref[:, 128:256] + agg_ref[1], w1_ref[128:256, :])
    t += b1_ref[...]
    t1_ref[...] = t
    _accum_stats(i, t, acc_ref, stats_ref)


def _k2_body(t1_ref, stats1_ref, g1_ref, bt1_ref, w2_ref, b2_ref,
             t2_ref, stats2_ref, acc_ref):
    i = pl.program_id(0)
    u = jnp.maximum(_bn_in(t1_ref[...], stats1_ref[...], g1_ref[...], bt1_ref[...]), 0.0)
    t = _dot(u, w2_ref[...]) + b2_ref[...]
    t2_ref[...] = t
    _accum_stats(i, t, acc_ref, stats2_ref)


def _k3_body(t2_ref, stats2_ref, gog_ref, gob_ref, hb_ref):
    h = jnp.maximum(_bn_in(t2_ref[...], stats2_ref[...], gog_ref[...], gob_ref[...]), 0.0)
    for b in range(4):
        hb_ref[b] = h[:, b * 128:(b + 1) * 128]


def _k4_body(hb_ref, agg_ref, w3_ref, b3_ref, t3_ref, stats3_ref, acc_ref):
    i = pl.program_id(0)
    t = b3_ref[...] + _dot(hb_ref[0] + agg_ref[0], w3_ref[0:128, :])
    for b in range(1, 4):
        t += _dot(hb_ref[b] + agg_ref[b], w3_ref[b * 128:(b + 1) * 128, :])
    t3_ref[...] = t
    _accum_stats(i, t, acc_ref, stats3_ref)


def _k5_body(t3_ref, stats3_ref, g2_ref, bt2_ref, w4_ref, b4_ref, h2_ref):
    v = jnp.maximum(_bn_in(t3_ref[...], stats3_ref[...], g2_ref[...], bt2_ref[...]), 0.0)
    h2_ref[...] = _dot(v, w4_ref[...]) + b4_ref[...]


def _k6_body(h2_ref, ctx_ref, wr1_ref, br1_ref, wr2_ref, br2_ref,
             wd_ref, bd_ref, wc_ref, bc_ref, out_ref, expl_ref):
    e = jnp.maximum(_dot(ctx_ref[...], wr1_ref[...]) + br1_ref[...], 0.0)
    expl = _dot(e, wr2_ref[...]) + br2_ref[...]
    expl_ref[...] = expl
    recon = _dot(expl, wd_ref[...]) + bd_ref[...]
    o = _dot(h2_ref[...] + 0.1 * recon, wc_ref[...]) + bc_ref[...]
    m = jnp.max(o, axis=1, keepdims=True)
    lse = jnp.log(jnp.sum(jnp.exp(o - m), axis=1, keepdims=True)) + m
    out_ref[...] = o - lse


def _row_spec(cols):
    return pl.BlockSpec((ROW_BLK, cols), lambda i: (i, 0))


def _blk_spec(nblk):
    return pl.BlockSpec((nblk, ROW_BLK, 128), lambda i: (0, i, 0))


def _full_spec(r, c):
    return pl.BlockSpec((r, c), lambda i: (0, 0))


_STATS = jax.ShapeDtypeStruct((2, H), jnp.float32)
_SEQ = pltpu.CompilerParams(dimension_semantics=("arbitrary",))


def _call(body, in_specs, out_specs, out_shape, scratch=False):
    return pl.pallas_call(
        body,
        grid=(GRID,),
        in_specs=in_specs,
        out_specs=out_specs,
        out_shape=out_shape,
        scratch_shapes=[pltpu.VMEM((2, H), jnp.float32)] if scratch else [],
        compiler_params=_SEQ,
    )


def kernel(x, edge_index, context, W1, b1, g1, bt1, W2, b2, go_g, go_b,
           W3, b3, g2, bt2, W4, b4, Wr1, br1, Wr2, br2, Wd, bd, Wc, bc):
    f32 = jnp.float32
    src = edge_index[0]
    dst = edge_index[1]
    pad = E_PAD - E
    srcp = jnp.concatenate([src, jnp.zeros((pad,), jnp.int32)])
    dstp = jnp.concatenate([dst, jnp.full((pad,), N, jnp.int32)])
    zeros = jnp.zeros((ZROWS, 128), f32)

    xb = x.reshape(N, 2, 128).transpose(1, 0, 2).reshape(2 * N, 128)

    agg1 = _get_segsum(2)(xb, srcp, dstp, zeros).reshape(2, N, 128)

    row1 = lambda v: v.reshape(1, -1)

    t1, stats1 = _call(
        _k1_body,
        [_row_spec(D_IN), _blk_spec(2), _full_spec(D_IN, H), _full_spec(1, H)],
        [_row_spec(H), _full_spec(2, H)],
        [jax.ShapeDtypeStruct((N, H), f32), _STATS],
        scratch=True,
    )(x, agg1, W1, row1(b1))

    t2, stats2 = _call(
        _k2_body,
        [_row_spec(H), _full_spec(2, H), _full_spec(1, H), _full_spec(1, H),
         _full_spec(H, H), _full_spec(1, H)],
        [_row_spec(H), _full_spec(2, H)],
        [jax.ShapeDtypeStruct((N, H), f32), _STATS],
        scratch=True,
    )(t1, stats1, row1(g1), row1(bt1), W2, row1(b2))

    (hb,) = _call(
        _k3_body,
        [_row_spec(H), _full_spec(2, H), _full_spec(1, H), _full_spec(1, H)],
        [_blk_spec(4)],
        [jax.ShapeDtypeStruct((4, N, 128), f32)],
    )(t2, stats2, row1(go_g), row1(go_b))

    agg2 = _get_segsum(4)(hb.reshape(4 * N, 128), srcp, dstp, zeros).reshape(4, N, 128)

    t3, stats3 = _call(
        _k4_body,
        [_blk_spec(4), _blk_spec(4), _full_spec(H, H), _full_spec(1, H)],
        [_row_spec(H), _full_spec(2, H)],
        [jax.ShapeDtypeStruct((N, H), f32), _STATS],
        scratch=True,
    )(hb, agg2, W3, row1(b3))

    (h2,) = _call(
        _k5_body,
        [_row_spec(H), _full_spec(2, H), _full_spec(1, H), _full_spec(1, H),
         _full_spec(H, H), _full_spec(1, H)],
        [_row_spec(H)],
        [jax.ShapeDtypeStruct((N, H), f32)],
    )(t3, stats3, row1(g2), row1(bt2), W4, row1(b4))

    ctx = jnp.pad(context, ((0, 0), (0, CTX_PAD - context.shape[1])))
    wr1 = jnp.pad(Wr1, ((0, CTX_PAD - Wr1.shape[0]), (0, 0)))

    out, expl = _call(
        _k6_body,
        [_row_spec(H), _row_spec(CTX_PAD), _full_spec(CTX_PAD, H),
         _full_spec(1, H), _full_spec(H, EXPL), _full_spec(1, EXPL),
         _full_spec(EXPL, H), _full_spec(1, H), _full_spec(H, C_OUT),
         _full_spec(1, C_OUT)],
        [_row_spec(C_OUT), _row_spec(EXPL)],
        [jax.ShapeDtypeStruct((N, C_OUT), f32),
         jax.ShapeDtypeStruct((N, EXPL), f32)],
    )(h2, ctx, wr1, row1(br1), Wr2, row1(br2), Wd, row1(bd), Wc, row1(bc))

    return (out, expl)


# speedup vs baseline: 2.6239x; 1.0542x over previous
import functools

import jax
import jax.numpy as jnp
from jax import lax
from jax.experimental import pallas as pl
from jax.experimental.pallas import tpu as pltpu
from jax.experimental.pallas import tpu_sc as plsc

N = 10000
E = 160000
D_IN = 256
H = 512
C_OUT = 64
CTX_PAD = 8
EXPL = 32

LANES = 16
NUM_TILES = 16
K_EDGE = 128
CH = 80
E_PAD = NUM_TILES * K_EDGE * CH
EPT = K_EDGE * CH
HCH = CH // 2
ACC_ROWS = 10240
ZROWS = ACC_ROWS // NUM_TILES
WROWS = 624
WCHUNKS = ((0, 128), (128, 128), (256, 128), (384, 128), (512, 112))
WTAIL = N - NUM_TILES * WROWS

ROW_BLK = 400
GRID = N // ROW_BLK


def _make_segsum(nblk):
    bpc = nblk // 2

    mesh = plsc.VectorSubcoreMesh(core_axis_name="c", subcore_axis_name="s")

    @functools.partial(
        pl.kernel,
        mesh=mesh,
        out_type=jax.ShapeDtypeStruct((nblk * N, 128), jnp.float32),
        scratch_types=[
            pltpu.VMEM((EPT // 2,), jnp.int32),
            pltpu.VMEM((HCH, K_EDGE), jnp.int32),
            pltpu.VMEM((K_EDGE, 128), jnp.float32),
            pltpu.VMEM((K_EDGE, 128), jnp.float32),
            pltpu.VMEM_SHARED((ACC_ROWS, 128), jnp.float32),
            pltpu.SemaphoreType.DMA,
            pltpu.SemaphoreType.DMA,
        ],
    )
    def segsum(x_hbm, sadj_hbm, dst_hbm, zeros_hbm, out_hbm,
               sadj, dst2, buf_a, buf_b, acc, sem_a, sem_b):
        c = lax.axis_index("c")
        s = lax.axis_index("s")

        def gather(ci, buf, sem):
            return pltpu.async_copy(
                x_hbm.at[sadj.at[pl.ds(ci * K_EDGE, K_EDGE)]], buf, sem)

        def wait(buf, sem):
            pltpu.make_async_copy(x_hbm.at[pl.ds(0, K_EDGE)], buf, sem).wait()

        def scatter(ci, buf):
            pltpu.sync_copy(buf, acc.at[dst2.at[ci]], add=True)

        for local in range(bpc):
            b = c * bpc + local
            row_off = b * N

            pltpu.sync_copy(zeros_hbm, acc.at[pl.ds(s * ZROWS, ZROWS)])
            plsc.subcore_barrier()

            for half in range(2):
                ibase = b * E_PAD + s * EPT + half * (EPT // 2)
                pltpu.sync_copy(sadj_hbm.at[pl.ds(ibase, EPT // 2)], sadj)
                pltpu.sync_copy(dst_hbm.at[s, pl.ds(half * HCH, HCH)], dst2)

                gather(0, buf_a, sem_a)

                def edge_pair(i, carry):
                    gather(2 * i + 1, buf_b, sem_b)
                    wait(buf_a, sem_a)
                    scatter(2 * i, buf_a)
                    gather(lax.min(2 * i + 2, HCH - 1), buf_a, sem_a)
                    wait(buf_b, sem_b)
                    scatter(2 * i + 1, buf_b)
                    return carry

                lax.fori_loop(0, HCH // 2, edge_pair, 0)
                wait(buf_a, sem_a)
            plsc.subcore_barrier()

            for off, sz in WCHUNKS:
                r0 = s * WROWS + off
                pltpu.sync_copy(acc.at[pl.ds(r0, sz)], buf_a.at[pl.ds(0, sz)])
                pltpu.sync_copy(buf_a.at[pl.ds(0, sz)],
                                out_hbm.at[pl.ds(row_off + r0, sz)])

            @pl.when(s == NUM_TILES - 1)
            def _():
                r0 = NUM_TILES * WROWS
                pltpu.sync_copy(acc.at[pl.ds(r0, WTAIL)],
                                buf_a.at[pl.ds(0, WTAIL)])
                pltpu.sync_copy(buf_a.at[pl.ds(0, WTAIL)],
                                out_hbm.at[pl.ds(row_off + r0, WTAIL)])
            if local + 1 < bpc:
                plsc.subcore_barrier()

    return segsum


@functools.lru_cache(maxsize=None)
def _get_segsum(nblk):
    return _make_segsum(nblk)


def _dot(a, b):
    return jnp.dot(a, b, preferred_element_type=jnp.float32)


def _accum_stats(i, t, acc_ref, stats_ref):
    @pl.when(i == 0)
    def _():
        acc_ref[...] = jnp.zeros_like(acc_ref)

    acc_ref[0:1, :] += jnp.sum(t, axis=0, keepdims=True)
    acc_ref[1:2, :] += jnp.sum(t * t, axis=0, keepdims=True)

    @pl.when(i == pl.num_programs(0) - 1)
    def _():
        stats_ref[...] = acc_ref[...]


def _bn_in(t, stats, g, bt):
    mu = stats[0:1, :] / N
    var = stats[1:2, :] / N - mu * mu
    inv = lax.rsqrt(var + 1e-5)
    return (t - mu) * inv * g + bt


def _k1_body(x_ref, agg_ref, w1_ref, b1_ref, t1_ref, stats_ref, acc_ref):
    i = pl.program_id(0)
    t = _dot(x_ref[:, 0:128] + agg_ref[0], w1_ref[0:128, :])
    t += _dot(x_ref[:, 128:256] + agg_ref[1], w1_ref[128:256, :])
    t += b1_ref[...]
    t1_ref[...] = t
    _accum_stats(i, t, acc_ref, stats_ref)


def _k2_body(t1_ref, stats1_ref, g1_ref, bt1_ref, w2_ref, b2_ref,
             t2_ref, stats2_ref, acc_ref):
    i = pl.program_id(0)
    u = jnp.maximum(_bn_in(t1_ref[...], stats1_ref[...], g1_ref[...], bt1_ref[...]), 0.0)
    t = _dot(u, w2_ref[...]) + b2_ref[...]
    t2_ref[...] = t
    _accum_stats(i, t, acc_ref, stats2_ref)


def _k3_body(t2_ref, stats2_ref, gog_ref, gob_ref, hb_ref):
    h = jnp.maximum(_bn_in(t2_ref[...], stats2_ref[...], gog_ref[...], gob_ref[...]), 0.0)
    for b in range(4):
        hb_ref[b] = h[:, b * 128:(b + 1) * 128]


def _k4_body(hb_ref, agg_ref, w3_ref, b3_ref, t3_ref, stats3_ref, acc_ref):
    i = pl.program_id(0)
    t = b3_ref[...] + _dot(hb_ref[0] + agg_ref[0], w3_ref[0:128, :])
    for b in range(1, 4):
        t += _dot(hb_ref[b] + agg_ref[b], w3_ref[b * 128:(b + 1) * 128, :])
    t3_ref[...] = t
    _accum_stats(i, t, acc_ref, stats3_ref)


def _k5_body(t3_ref, stats3_ref, g2_ref, bt2_ref, w4_ref, b4_ref, h2_ref):
    v = jnp.maximum(_bn_in(t3_ref[...], stats3_ref[...], g2_ref[...], bt2_ref[...]), 0.0)
    h2_ref[...] = _dot(v, w4_ref[...]) + b4_ref[...]


def _k6_body(h2_ref, ctx_ref, wr1_ref, br1_ref, wr2_ref, br2_ref,
             wd_ref, bd_ref, wc_ref, bc_ref, out_ref, expl_ref):
    e = jnp.maximum(_dot(ctx_ref[...], wr1_ref[...]) + br1_ref[...], 0.0)
    expl = _dot(e, wr2_ref[...]) + br2_ref[...]
    expl_ref[...] = expl
    recon = _dot(expl, wd_ref[...]) + bd_ref[...]
    o = _dot(h2_ref[...] + 0.1 * recon, wc_ref[...]) + bc_ref[...]
    m = jnp.max(o, axis=1, keepdims=True)
    lse = jnp.log(jnp.sum(jnp.exp(o - m), axis=1, keepdims=True)) + m
    out_ref[...] = o - lse


def _row_spec(cols):
    return pl.BlockSpec((ROW_BLK, cols), lambda i: (i, 0))


def _blk_spec(nblk):
    return pl.BlockSpec((nblk, ROW_BLK, 128), lambda i: (0, i, 0))


def _full_spec(r, c):
    return pl.BlockSpec((r, c), lambda i: (0, 0))


_STATS = jax.ShapeDtypeStruct((2, H), jnp.float32)
_SEQ = pltpu.CompilerParams(dimension_semantics=("arbitrary",))


def _call(body, in_specs, out_specs, out_shape, scratch=False):
    return pl.pallas_call(
        body,
        grid=(GRID,),
        in_specs=in_specs,
        out_specs=out_specs,
        out_shape=out_shape,
        scratch_shapes=[pltpu.VMEM((2, H), jnp.float32)] if scratch else [],
        compiler_params=_SEQ,
    )


def kernel(x, edge_index, context, W1, b1, g1, bt1, W2, b2, go_g, go_b,
           W3, b3, g2, bt2, W4, b4, Wr1, br1, Wr2, br2, Wd, bd, Wc, bc):
    f32 = jnp.float32
    src = edge_index[0]
    dst = edge_index[1]
    pad = E_PAD - E
    srcp = jnp.concatenate([src, jnp.zeros((pad,), jnp.int32)])
    dstp = jnp.concatenate([dst, jnp.full((pad,), N, jnp.int32)])
    dstp = dstp.reshape(NUM_TILES, CH, K_EDGE)
    off2 = (jnp.arange(2, dtype=jnp.int32) * N)[:, None]
    off4 = (jnp.arange(4, dtype=jnp.int32) * N)[:, None]
    sadj2 = (srcp[None, :] + off2).reshape(-1)
    sadj4 = (srcp[None, :] + off4).reshape(-1)
    zeros = jnp.zeros((ZROWS, 128), f32)

    xb = x.reshape(N, 2, 128).transpose(1, 0, 2).reshape(2 * N, 128)

    agg1 = _get_segsum(2)(xb, sadj2, dstp, zeros).reshape(2, N, 128)

    row1 = lambda v: v.reshape(1, -1)

    t1, stats1 = _call(
        _k1_body,
        [_row_spec(D_IN), _blk_spec(2), _full_spec(D_IN, H), _full_spec(1, H)],
        [_row_spec(H), _full_spec(2, H)],
        [jax.ShapeDtypeStruct((N, H), f32), _STATS],
        scratch=True,
    )(x, agg1, W1, row1(b1))

    t2, stats2 = _call(
        _k2_body,
        [_row_spec(H), _full_spec(2, H), _full_spec(1, H), _full_spec(1, H),
         _full_spec(H, H), _full_spec(1, H)],
        [_row_spec(H), _full_spec(2, H)],
        [jax.ShapeDtypeStruct((N, H), f32), _STATS],
        scratch=True,
    )(t1, stats1, row1(g1), row1(bt1), W2, row1(b2))

    (hb,) = _call(
        _k3_body,
        [_row_spec(H), _full_spec(2, H), _full_spec(1, H), _full_spec(1, H)],
        [_blk_spec(4)],
        [jax.ShapeDtypeStruct((4, N, 128), f32)],
    )(t2, stats2, row1(go_g), row1(go_b))

    agg2 = _get_segsum(4)(hb.reshape(4 * N, 128), sadj4, dstp, zeros).reshape(4, N, 128)

    t3, stats3 = _call(
        _k4_body,
        [_blk_spec(4), _blk_spec(4), _full_spec(H, H), _full_spec(1, H)],
        [_row_spec(H), _full_spec(2, H)],
        [jax.ShapeDtypeStruct((N, H), f32), _STATS],
        scratch=True,
    )(hb, agg2, W3, row1(b3))

    (h2,) = _call(
        _k5_body,
        [_row_spec(H), _full_spec(2, H), _full_spec(1, H), _full_spec(1, H),
         _full_spec(H, H), _full_spec(1, H)],
        [_row_spec(H)],
        [jax.ShapeDtypeStruct((N, H), f32)],
    )(t3, stats3, row1(g2), row1(bt2), W4, row1(b4))

    ctx = jnp.pad(context, ((0, 0), (0, CTX_PAD - context.shape[1])))
    wr1 = jnp.pad(Wr1, ((0, CTX_PAD - Wr1.shape[0]), (0, 0)))

    out, expl = _call(
        _k6_body,
        [_row_spec(H), _row_spec(CTX_PAD), _full_spec(CTX_PAD, H),
         _full_spec(1, H), _full_spec(H, EXPL), _full_spec(1, EXPL),
         _full_spec(EXPL, H), _full_spec(1, H), _full_spec(H, C_OUT),
         _full_spec(1, C_OUT)],
        [_row_spec(C_OUT), _row_spec(EXPL)],
        [jax.ShapeDtypeStruct((N, C_OUT), f32),
         jax.ShapeDtypeStruct((N, EXPL), f32)],
    )(h2, ctx, wr1, row1(br1), Wr2, row1(br2), Wd, row1(bd), Wc, row1(bc))

    return (out, expl)
